# baseline (speedup 1.0000x reference)
import jax
import jax.numpy as jnp
from jax.experimental import pallas as pl
from jax.experimental.pallas import tpu as pltpu

_T = 100
_BBLK = 16
_R = 4


def _body(x_ref, d_ref, o_hbm, buf, sems):
    i = pl.program_id(0)
    n = pl.num_programs(0)
    slot = jax.lax.rem(i, _R)

    @pl.when(i >= _R)
    def _wait_prev():
        pltpu.make_async_copy(
            buf.at[slot],
            o_hbm.at[pl.ds((i - _R) * _BBLK, _BBLK)],
            sems.at[slot],
        ).wait()

    st = ((1.0 - x_ref[...] * d_ref[...]) * (_T - 1)).astype(jnp.int32)
    st = jnp.clip(st, 0, _T - 1)
    t = jax.lax.broadcasted_iota(jnp.int32, (_BBLK, _T, st.shape[-1]), 1)
    buf[slot] = (t == st[:, None, :]).astype(jnp.float32)

    pltpu.make_async_copy(
        buf.at[slot],
        o_hbm.at[pl.ds(i * _BBLK, _BBLK)],
        sems.at[slot],
    ).start()

    @pl.when(i == n - 1)
    def _drain():
        for r in range(_R):
            j = n - _R + r

            @pl.when(j >= 0)
            def _():
                s = jax.lax.rem(j, _R)
                pltpu.make_async_copy(
                    buf.at[s],
                    o_hbm.at[pl.ds(j * _BBLK, _BBLK)],
                    sems.at[s],
                ).wait()


def kernel(x, delays):
    b, f = x.shape
    return pl.pallas_call(
        _body,
        grid=(b // _BBLK,),
        in_specs=[
            pl.BlockSpec((_BBLK, f), lambda i: (i, 0)),
            pl.BlockSpec((1, f), lambda i: (0, 0)),
        ],
        out_specs=pl.BlockSpec(memory_space=pltpu.MemorySpace.HBM),
        out_shape=jax.ShapeDtypeStruct((b, _T, f), jnp.float32),
        scratch_shapes=[
            pltpu.VMEM((_R, _BBLK, _T, f), jnp.float32),
            pltpu.SemaphoreType.DMA((_R,)),
        ],
    )(x, delays[None, :])
